# baseline (speedup 1.0000x reference)
import functools

import jax
import jax.numpy as jnp
from jax import lax
from jax.experimental import pallas as pl
from jax.experimental.pallas import tpu as pltpu
from jax.experimental.pallas import tpu_sc as plsc

NC = 2
NS = 16
NW = NC * NS


def _pick_chunk(ept: int) -> int:
    for c in range(10000, 7, -8):
        if ept % c == 0 and (ept // c) % 4 == 0:
            return c
    return 8

def _make_push_call(E: int, NPAD: int):
    assert E % NW == 0
    EPT = E // NW
    C = _pick_chunk(EPT)
    NCHUNK = EPT // C
    CHN = NPAD // NS
    assert NPAD % NS == 0 and CHN % 8 == 0

    mesh = plsc.VectorSubcoreMesh(core_axis_name="c", subcore_axis_name="s")

    @functools.partial(
        pl.kernel,
        out_type=jax.ShapeDtypeStruct((NC * NPAD,), jnp.float32),
        mesh=mesh,
        scratch_types=(
            [pltpu.VMEM((C,), jnp.int32)] * 4
            + [pltpu.VMEM((C,), jnp.int32)] * 4
            + [pltpu.VMEM((C,), jnp.float32)] * 2
            + [
                pltpu.VMEM_SHARED((NPAD,), jnp.float32),
                pltpu.VMEM_SHARED((NPAD,), jnp.float32),
                pltpu.VMEM((CHN,), jnp.float32),
            ]
            + [pltpu.SemaphoreType.DMA] * 8
        ),
    )
    def push(src_hbm, tgt_hbm, val_hbm, zero_hbm, out_hbm,
             is0, is1, is2, is3, it0, it1, it2, it3, vals0, vals1,
             accum, vtab, stage,
             si0, si1, si2, si3, sg0, sg1, ss0, ss1):
        idx_s = (is0, is1, is2, is3)
        idx_t = (it0, it1, it2, it3)
        vals = (vals0, vals1)
        sem_i = (si0, si1, si2, si3)
        sem = (sg0, sg1)
        sem_s = (ss0, ss1)
        c = lax.axis_index("c")
        s = lax.axis_index("s")
        wid = c * NS + s

        ebase = wid * EPT

        pltpu.sync_copy(val_hbm.at[pl.ds(s * CHN, CHN)], stage)
        pltpu.sync_copy(stage, vtab.at[pl.ds(s * CHN, CHN)])
        for k in range(4):
            off = ebase + k * C
            pltpu.async_copy(src_hbm.at[pl.ds(off, C)], idx_s[k], sem_i[k])
            pltpu.async_copy(tgt_hbm.at[pl.ds(off, C)], idx_t[k], sem_i[k])
        pltpu.sync_copy(zero_hbm.at[pl.ds(s * CHN, CHN)], stage)
        pltpu.sync_copy(stage, accum.at[pl.ds(s * CHN, CHN)])
        plsc.subcore_barrier()

        @pl.loop(0, NCHUNK, step=4)
        def _(g):
            for half in range(2):
                p = (2 * half, 2 * half + 1)
                gp = g + 2 * half

                for b in range(2):
                    off = ebase + (gp + b) * C
                    pltpu.make_async_copy(src_hbm.at[pl.ds(off, C)],
                                          idx_s[p[b]], sem_i[p[b]]).wait()
                    pltpu.make_async_copy(tgt_hbm.at[pl.ds(off, C)],
                                          idx_t[p[b]], sem_i[p[b]]).wait()
                    pltpu.async_copy(vtab.at[idx_s[p[b]]], vals[b], sem[b])
                for b in range(2):
                    pltpu.make_async_copy(vtab.at[idx_s[p[b]]], vals[b],
                                          sem[b]).wait()
                plsc.subcore_barrier()

                for b in range(2):
                    pltpu.async_copy(vals[b], accum.at[idx_t[p[b]]],
                                     sem_s[b], add=True)
                for b in range(2):
                    pltpu.make_async_copy(vals[b], accum.at[idx_t[p[b]]],
                                          sem_s[b]).wait()

                @pl.when(gp + 4 < NCHUNK)
                def _():
                    for b in range(2):
                        off2 = ebase + (gp + 4 + b) * C
                        pltpu.async_copy(src_hbm.at[pl.ds(off2, C)],
                                         idx_s[p[b]], sem_i[p[b]])
                        pltpu.async_copy(tgt_hbm.at[pl.ds(off2, C)],
                                         idx_t[p[b]], sem_i[p[b]])
                plsc.subcore_barrier()

        plsc.subcore_barrier()
        pltpu.sync_copy(accum.at[pl.ds(s * CHN, CHN)], stage)
        pltpu.sync_copy(stage, out_hbm.at[pl.ds(c * NPAD + s * CHN, CHN)])

    return push


def kernel(source, target, init_vertex, iteration, vertex_num):
    N = init_vertex.shape[0]
    E = source.shape[0]
    NPAD = -(-N // (NS * 8)) * (NS * 8)

    src = source.astype(jnp.int32)
    tgt = target.astype(jnp.int32)

    push = _make_push_call(E, NPAD)

    zeros_pad = jnp.zeros((NPAD,), jnp.float32)
    ones_pad = zeros_pad.at[:N].set(1.0)

    deg_parts = push(src, src, ones_pad, zeros_pad)
    deg = deg_parts[:N] + deg_parts[NPAD:NPAD + N]
    mask = deg == 0.0
    degf = jnp.where(mask, 1.0, deg)

    V0 = init_vertex / jnp.sum(init_vertex)

    def cond_fun(carry):
        r, V_old, done = carry
        return jnp.logical_and(r < iteration, jnp.logical_not(done))

    def body_fun(carry):
        r, V_old, done = carry
        vtemp = jnp.where(mask, 0.0, V_old / degf)
        blind_sum = jnp.sum(jnp.where(mask, V_old, 0.0))
        vtemp_pad = jnp.concatenate([vtemp, jnp.zeros((NPAD - N,), jnp.float32)])
        parts = push(src, tgt, vtemp_pad, zeros_pad)
        V_new = parts[:N] + parts[NPAD:NPAD + N]
        V_new = V_new * 0.85 + (0.15 + blind_sum * 0.85) / vertex_num
        diff = jnp.sum(jnp.abs(V_new - V_old))
        return (r + 1, V_new, diff < 1e-07)

    carry = (jnp.int32(0), V0, jnp.bool_(False))
    _, V_out, _ = lax.while_loop(cond_fun, body_fun, carry)
    return V_out
